# baseline (speedup 1.0000x reference)
import jax
import jax.numpy as jnp
from jax.experimental import pallas as pl
from jax.experimental.pallas import tpu as pltpu

NUM_BINS = 4
INTERVAL_SCALE = 1.0 / float(4 * (NUM_BINS - 2) + 2)
CC = 8


def _body(x_ref, w1_ref, b1_ref, w2_ref, b2_ref, out_ref, acc_ref, sem):
    B = pl.num_programs(0)
    b = pl.program_id(0)
    _, C, T, HW = x_ref.shape

    parts = []
    for g in range(C // CC):
        xg = x_ref[0, CC * g:CC * (g + 1)]
        parts.append(jnp.sum(xg, axis=2))
    pooled = jnp.concatenate(parts, axis=0) * (1.0 / HW)

    h = jnp.dot(w1_ref[...], pooled, preferred_element_type=jnp.float32)
    h = h + b1_ref[...]
    h = h * jnp.clip(h + 3.0, 0.0, 6.0) * (1.0 / 6.0)
    e = jnp.dot(w2_ref[...], h, preferred_element_type=jnp.float32)
    e = e + b2_ref[...]

    norms = jnp.sum(e * e, axis=0, keepdims=True)
    mn = jnp.min(norms, axis=1, keepdims=True)
    mx = jnp.max(norms, axis=1, keepdims=True)
    gamma = INTERVAL_SCALE * (mx - mn)
    cbuf = 4.0 * jax.lax.broadcasted_iota(
        jnp.int32, (NUM_BINS, 1), 0).astype(jnp.float32) - 1.0
    centers = mn + gamma * cbuf
    diff = norms - centers
    unscaled = jnp.maximum(1.0 - jnp.abs(diff) * (0.5 / gamma), 0.0)
    s = jnp.sum(unscaled, axis=1, keepdims=True)
    scales = jnp.where(s > 0.0, 1.0 / s, 1.0)
    coeff_nt = scales * unscaled

    tiled = jnp.tile(coeff_nt, (CC, CC))
    r_ch = jax.lax.broadcasted_iota(jnp.int32, (CC * NUM_BINS, CC * T), 0) // NUM_BINS
    k_ch = jax.lax.broadcasted_iota(jnp.int32, (CC * NUM_BINS, CC * T), 1) // T
    wbd = jnp.where(r_ch == k_ch, tiled, 0.0)

    @pl.when(b > 0)
    def _wait_prev():
        pltpu.make_async_copy(acc_ref, out_ref.at[b], sem).wait()

    for g in range(C // CC):
        xg2 = x_ref[0, CC * g:CC * (g + 1)].reshape(CC * T, HW)
        acc_ref[CC * NUM_BINS * g:CC * NUM_BINS * (g + 1)] = jnp.dot(
            wbd, xg2, preferred_element_type=jnp.float32)

    pltpu.make_async_copy(acc_ref, out_ref.at[b], sem).start()

    @pl.when(b == B - 1)
    def _wait_last():
        pltpu.make_async_copy(acc_ref, out_ref.at[b], sem).wait()


def kernel(x, w1, b1, w2, b2, *, interpret=False):
    B, C, T, H, W = x.shape
    HW = H * W
    HIDDEN = w1.shape[0]
    E = w2.shape[0]
    xr = x.reshape(B, C, T, HW)
    out = pl.pallas_call(
        _body,
        grid=(B,),
        in_specs=[
            pl.BlockSpec((1, C, T, HW), lambda b: (b, 0, 0, 0)),
            pl.BlockSpec((HIDDEN, C), lambda b: (0, 0)),
            pl.BlockSpec((HIDDEN, 1), lambda b: (0, 0)),
            pl.BlockSpec((E, HIDDEN), lambda b: (0, 0)),
            pl.BlockSpec((E, 1), lambda b: (0, 0)),
        ],
        out_specs=pl.BlockSpec(memory_space=pl.ANY),
        out_shape=jax.ShapeDtypeStruct((B, C * NUM_BINS, HW), jnp.float32),
        scratch_shapes=[
            pltpu.VMEM((C * NUM_BINS, HW), jnp.float32),
            pltpu.SemaphoreType.DMA,
        ],
        compiler_params=pltpu.CompilerParams(
            dimension_semantics=("arbitrary",),
            vmem_limit_bytes=62 * 1024 * 1024,
        ),
        name="sgs_fused",
        interpret=interpret,
    )(xr, w1, b1.reshape(HIDDEN, 1), w2, b2.reshape(E, 1))
    return out.reshape(B, C, NUM_BINS, H, W)


# speedup vs baseline: 7.0161x; 7.0161x over previous
import jax
import jax.numpy as jnp
from jax.experimental import pallas as pl
from jax.experimental.pallas import tpu as pltpu

NUM_BINS = 4
INTERVAL_SCALE = 1.0 / float(4 * (NUM_BINS - 2) + 2)
HWC = 8
PG = 8


def _body(x_ref, w1_ref, b1_ref, w2_ref, b2_ref, out_ref, acc_ref, sem):
    B = pl.num_programs(0)
    b = pl.program_id(0)
    _, HW, T, C = x_ref.shape
    CL = 128
    NCT = C // CL
    step = HW // HWC
    R = NUM_BINS * PG

    psum = None
    for i in range(HWC):
        part = jnp.sum(x_ref[0, step * i:step * (i + 1)], axis=0)
        psum = part if psum is None else psum + part
    pooled = psum * (1.0 / HW)

    h = jax.lax.dot_general(pooled, w1_ref[...], (((1,), (1,)), ((), ())),
                            preferred_element_type=jnp.float32)
    h = h + b1_ref[...]
    h = h * jnp.clip(h + 3.0, 0.0, 6.0) * (1.0 / 6.0)
    e = jax.lax.dot_general(h, w2_ref[...], (((1,), (1,)), ((), ())),
                            preferred_element_type=jnp.float32)
    e = e + b2_ref[...]

    norms = jnp.sum(e * e, axis=1, keepdims=True)
    mn = jnp.min(norms, axis=0, keepdims=True)
    mx = jnp.max(norms, axis=0, keepdims=True)
    gamma = INTERVAL_SCALE * (mx - mn)
    cbuf = 4.0 * jax.lax.broadcasted_iota(
        jnp.int32, (1, NUM_BINS), 1).astype(jnp.float32) - 1.0
    centers = mn + gamma * cbuf
    diff = norms - centers
    unscaled = jnp.maximum(1.0 - jnp.abs(diff) * (0.5 / gamma), 0.0)
    s = jnp.sum(unscaled, axis=0, keepdims=True)
    scales = jnp.where(s > 0.0, 1.0 / s, 1.0)
    coeff_tn = scales * unscaled

    l_rn = jnp.where(
        jax.lax.broadcasted_iota(jnp.int32, (R, NUM_BINS), 0) // PG
        == jax.lax.broadcasted_iota(jnp.int32, (R, NUM_BINS), 1),
        1.0, 0.0)
    g = jax.lax.dot_general(l_rn, coeff_tn, (((1,), (1,)), ((), ())),
                            preferred_element_type=jnp.float32)
    a = jnp.tile(g, (1, PG))
    p_match = (
        jax.lax.broadcasted_iota(jnp.int32, (R, PG * T), 0) % PG
        == jax.lax.broadcasted_iota(jnp.int32, (R, PG * T), 1) // T)
    wbd = jnp.where(p_match, a, 0.0)

    def _copies():
        for n in range(NUM_BINS):
            for ct in range(NCT):
                yield pltpu.make_async_copy(
                    acc_ref.at[n, ct],
                    out_ref.at[b, :, ct * NUM_BINS + n, :], sem)

    @pl.when(b > 0)
    def _wait_prev():
        for cp in _copies():
            cp.wait()

    for i in range(HW // PG):
        xg = x_ref[0, PG * i:PG * (i + 1)].reshape(PG * T, C)
        o8 = jnp.dot(wbd, xg, preferred_element_type=jnp.float32)
        for n in range(NUM_BINS):
            for ct in range(NCT):
                acc_ref[n, ct, PG * i:PG * (i + 1)] = \
                    o8[PG * n:PG * (n + 1), CL * ct:CL * (ct + 1)]

    for cp in _copies():
        cp.start()

    @pl.when(b == B - 1)
    def _wait_last():
        for cp in _copies():
            cp.wait()


def kernel(x, w1, b1, w2, b2, *, interpret=False):
    B, C, T, H, W = x.shape
    HW = H * W
    HIDDEN = w1.shape[0]
    E = w2.shape[0]
    CL = 128
    NCT = C // CL
    xt = x.transpose(0, 3, 4, 2, 1).reshape(B, HW, T, C)
    out = pl.pallas_call(
        _body,
        grid=(B,),
        in_specs=[
            pl.BlockSpec((1, HW, T, C), lambda b: (b, 0, 0, 0)),
            pl.BlockSpec((HIDDEN, C), lambda b: (0, 0)),
            pl.BlockSpec((1, HIDDEN), lambda b: (0, 0)),
            pl.BlockSpec((E, HIDDEN), lambda b: (0, 0)),
            pl.BlockSpec((1, E), lambda b: (0, 0)),
        ],
        out_specs=pl.BlockSpec(memory_space=pl.ANY),
        out_shape=jax.ShapeDtypeStruct((B, HW, NCT * NUM_BINS, CL), jnp.float32),
        scratch_shapes=[
            pltpu.VMEM((NUM_BINS, NCT, HW, CL), jnp.float32),
            pltpu.SemaphoreType.DMA,
        ],
        compiler_params=pltpu.CompilerParams(
            dimension_semantics=("arbitrary",),
            vmem_limit_bytes=62 * 1024 * 1024,
        ),
        name="sgs_fused",
        interpret=interpret,
    )(xt, w1, b1.reshape(1, HIDDEN), w2, b2.reshape(1, E))
    return (out.reshape(B, H, W, NCT, NUM_BINS, CL)
            .transpose(0, 3, 5, 4, 1, 2)
            .reshape(B, C, NUM_BINS, H, W))
